# baseline (speedup 1.0000x reference)
import functools

import jax
import jax.numpy as jnp
from jax import lax
from jax.experimental import pallas as pl
from jax.experimental.pallas import tpu as pltpu
from jax.experimental.pallas import tpu_sc as plsc

B, S, D = 4, 2048, 128
NC, NS, L = 2, 16, 16
NW = NC * NS
BPW = (B * S) // NW
NCH = 8
CH = BPW // NCH


def _emb_body(tok_hbm, emb_hbm, pos_hbm, out_hbm, idx_v, rows_v,
              p0, p1, p2, p3, p4, p5, p6, p7,
              g0, g1, g2, g3, g4, g5, g6, g7, ssem):
    wid = lax.axis_index("s") * NC + lax.axis_index("c")
    base = wid * BPW
    pos_start = lax.rem(base, S)
    psems = (p0, p1, p2, p3, p4, p5, p6, p7)
    gsems = (g0, g1, g2, g3, g4, g5, g6, g7)

    pcopies = [
        pltpu.async_copy(
            pos_hbm.at[pl.ds(pos_start + j * CH, CH)],
            rows_v.at[pl.ds(j * CH, CH)],
            psems[j],
        )
        for j in range(NCH)
    ]
    pltpu.sync_copy(tok_hbm.at[wid], idx_v)

    gadds = []
    for j in range(NCH):
        pcopies[j].wait()
        gadds.append(
            pltpu.async_copy(
                emb_hbm.at[idx_v.at[j]],
                rows_v.at[pl.ds(j * CH, CH)],
                gsems[j],
                add=True,
            )
        )
    stores = []
    for j in range(NCH):
        gadds[j].wait()
        stores.append(
            pltpu.async_copy(
                rows_v.at[pl.ds(j * CH, CH)],
                out_hbm.at[pl.ds(base + j * CH, CH)],
                ssem,
            )
        )
    for st in stores:
        st.wait()


@jax.jit
def _emb_call(tokens_flat, emb_table, pos_table):
    mesh = plsc.VectorSubcoreMesh(core_axis_name="c", subcore_axis_name="s")
    call = functools.partial(
        pl.kernel,
        mesh=mesh,
        out_type=jax.ShapeDtypeStruct((B * S, D), jnp.float32),
        scratch_types=[
            pltpu.VMEM((NCH, CH), jnp.int32),
            pltpu.VMEM((BPW, D), jnp.float32),
        ] + [pltpu.SemaphoreType.DMA] * 17,
    )(_emb_body)
    return call(tokens_flat, emb_table, pos_table)


def kernel(tokens, emb_table, pos_table):
    tokens_flat = tokens.astype(jnp.int32).reshape(NW, NCH, CH)
    out = _emb_call(tokens_flat, emb_table, pos_table)
    return out.reshape(B, S, D)


# speedup vs baseline: 1.0223x; 1.0223x over previous
import functools

import jax
import jax.numpy as jnp
from jax import lax
from jax.experimental import pallas as pl
from jax.experimental.pallas import tpu as pltpu
from jax.experimental.pallas import tpu_sc as plsc

B, S, D = 4, 2048, 128
NC, NS, L = 2, 16, 16
NW = NC * NS
BPW = (B * S) // NW
CHUNKS = (128, 96, 32)
OFFS = (0, 128, 224)


def _emb_body(tok_hbm, emb_hbm, pos_hbm, out_hbm, idx_v, rows_v,
              p0, p1, p2, g0, g1, g2, ssem):
    wid = lax.axis_index("s") * NC + lax.axis_index("c")
    base = wid * BPW
    pos_start = lax.rem(base, S)
    psems = (p0, p1, p2)
    gsems = (g0, g1, g2)

    pcopies = [
        pltpu.async_copy(
            pos_hbm.at[pl.ds(pos_start + off, ch)],
            rows_v.at[pl.ds(off, ch)],
            psems[j],
        )
        for j, (off, ch) in enumerate(zip(OFFS, CHUNKS))
    ]
    pltpu.sync_copy(tok_hbm.at[wid], idx_v)

    gadds = []
    for j, (off, ch) in enumerate(zip(OFFS, CHUNKS)):
        pcopies[j].wait()
        gadds.append(
            pltpu.async_copy(
                emb_hbm.at[idx_v.at[pl.ds(off, ch)]],
                rows_v.at[pl.ds(off, ch)],
                gsems[j],
                add=True,
            )
        )
    stores = []
    for j, (off, ch) in enumerate(zip(OFFS, CHUNKS)):
        gadds[j].wait()
        stores.append(
            pltpu.async_copy(
                rows_v.at[pl.ds(off, ch)],
                out_hbm.at[pl.ds(base + off, ch)],
                ssem,
            )
        )
    for st in stores:
        st.wait()


@jax.jit
def _emb_call(tokens_flat, emb_table, pos_table):
    mesh = plsc.VectorSubcoreMesh(core_axis_name="c", subcore_axis_name="s")
    call = functools.partial(
        pl.kernel,
        mesh=mesh,
        out_type=jax.ShapeDtypeStruct((B * S, D), jnp.float32),
        scratch_types=[
            pltpu.VMEM((BPW,), jnp.int32),
            pltpu.VMEM((BPW, D), jnp.float32),
        ] + [pltpu.SemaphoreType.DMA] * 7,
    )(_emb_body)
    return call(tokens_flat, emb_table, pos_table)


def kernel(tokens, emb_table, pos_table):
    tokens_flat = tokens.astype(jnp.int32).reshape(NW, BPW)
    out = _emb_call(tokens_flat, emb_table, pos_table)
    return out.reshape(B, S, D)


# speedup vs baseline: 1.0380x; 1.0154x over previous
import functools

import jax
import jax.numpy as jnp
from jax import lax
from jax.experimental import pallas as pl
from jax.experimental.pallas import tpu as pltpu
from jax.experimental.pallas import tpu_sc as plsc

B, S, D = 4, 2048, 128
NC, NS, L = 2, 16, 16
NW = NC * NS
BPW = (B * S) // NW
NCH = 2
CH = BPW // NCH


def _emb_body(tok_hbm, emb_hbm, pos_hbm, out_hbm, idx_v, rows_v,
              p0, p1, g0, g1, ssem, isem):
    wid = lax.axis_index("s") * NC + lax.axis_index("c")
    base = wid * BPW
    pos_start = lax.rem(base, S)
    psems = (p0, p1)
    gsems = (g0, g1)

    icopy = pltpu.async_copy(tok_hbm.at[wid], idx_v, isem)
    pcopies = [
        pltpu.async_copy(
            pos_hbm.at[pl.ds(pos_start + j * CH, CH)],
            rows_v.at[pl.ds(j * CH, CH)],
            psems[j],
        )
        for j in range(NCH)
    ]
    icopy.wait()

    gadds = []
    for j in range(NCH):
        pcopies[j].wait()
        gadds.append(
            pltpu.async_copy(
                emb_hbm.at[idx_v.at[j]],
                rows_v.at[pl.ds(j * CH, CH)],
                gsems[j],
                add=True,
            )
        )
    stores = []
    for j in range(NCH):
        gadds[j].wait()
        stores.append(
            pltpu.async_copy(
                rows_v.at[pl.ds(j * CH, CH)],
                out_hbm.at[pl.ds(base + j * CH, CH)],
                ssem,
            )
        )
    for st in stores:
        st.wait()


@jax.jit
def _emb_call(tokens_flat, emb_table, pos_table):
    mesh = plsc.VectorSubcoreMesh(core_axis_name="c", subcore_axis_name="s")
    call = functools.partial(
        pl.kernel,
        mesh=mesh,
        out_type=jax.ShapeDtypeStruct((B * S, D), jnp.float32),
        scratch_types=[
            pltpu.VMEM((NCH, CH), jnp.int32),
            pltpu.VMEM((BPW, D), jnp.float32),
        ] + [pltpu.SemaphoreType.DMA] * 6,
    )(_emb_body)
    return call(tokens_flat, emb_table, pos_table)


def kernel(tokens, emb_table, pos_table):
    tokens_flat = tokens.astype(jnp.int32).reshape(NW, NCH, CH)
    out = _emb_call(tokens_flat, emb_table, pos_table)
    return out.reshape(B, S, D)
